# baseline (speedup 1.0000x reference)
import functools

import jax
import jax.numpy as jnp
from jax import lax
from jax.experimental import pallas as pl
from jax.experimental.pallas import tpu as pltpu
from jax.experimental.pallas import tpu_sc as plsc

NC = 2
NS = 16
K = 128


def _make_sc_agg(n, hd, n_chunks, acc_rows):
  z_rows = acc_rows // NS

  mesh = plsc.VectorSubcoreMesh(core_axis_name="c", subcore_axis_name="s")

  @functools.partial(
      pl.kernel,
      out_type=jax.ShapeDtypeStruct((NC, acc_rows, hd), jnp.float32),
      mesh=mesh,
      scratch_types=[
          pltpu.VMEM((n_chunks + 1, K), jnp.int32),
          pltpu.VMEM((1, K), jnp.int32),
          pltpu.VMEM((1, K), jnp.int32),
          pltpu.VMEM((K, hd), jnp.float32),
          pltpu.VMEM((K, hd), jnp.float32),
          pltpu.SemaphoreType.DMA,
          pltpu.SemaphoreType.DMA,
          pltpu.SemaphoreType.DMA,
          pltpu.SemaphoreType.DMA,
          pltpu.VMEM_SHARED((acc_rows, hd), jnp.float32),
      ],
  )
  def sc_agg(xs_hbm, src_hbm, dst_hbm, zeros_hbm, out_hbm,
             srcv, d0, d1, g0, g1, semd0, semd1, semg0, semg1, acc):
    c = lax.axis_index("c")
    s = lax.axis_index("s")
    pltpu.sync_copy(zeros_hbm, acc.at[pl.ds(s * z_rows, z_rows)])
    pltpu.sync_copy(src_hbm.at[c, s], srcv)
    plsc.subcore_barrier()

    pltpu.async_copy(dst_hbm.at[s, 0], d0, semd0)
    pltpu.async_copy(xs_hbm.at[srcv.at[0]], g0, semg0)
    pltpu.async_copy(dst_hbm.at[s, 1], d1, semd1)

    def chunk_pair(p, carry):
      j0 = 2 * p
      pltpu.make_async_copy(xs_hbm.at[srcv.at[j0]], g0, semg0).wait()
      pltpu.async_copy(xs_hbm.at[srcv.at[j0 + 1]], g1, semg1)
      pltpu.make_async_copy(dst_hbm.at[s, j0], d0, semd0).wait()
      pltpu.sync_copy(g0, acc.at[d0.at[0]], add=True)
      pltpu.async_copy(dst_hbm.at[s, j0 + 2], d0, semd0)
      pltpu.make_async_copy(xs_hbm.at[srcv.at[j0 + 1]], g1, semg1).wait()
      pltpu.async_copy(xs_hbm.at[srcv.at[j0 + 2]], g0, semg0)
      pltpu.make_async_copy(dst_hbm.at[s, j0 + 1], d1, semd1).wait()
      pltpu.sync_copy(g1, acc.at[d1.at[0]], add=True)
      pltpu.async_copy(dst_hbm.at[s, j0 + 3], d1, semd1)
      return carry

    lax.fori_loop(0, n_chunks // 2, chunk_pair, 0, unroll=False)
    pltpu.make_async_copy(xs_hbm.at[srcv.at[0]], g0, semg0).wait()
    pltpu.make_async_copy(dst_hbm.at[s, 0], d0, semd0).wait()
    pltpu.make_async_copy(dst_hbm.at[s, 1], d1, semd1).wait()
    plsc.subcore_barrier()
    pltpu.sync_copy(acc.at[pl.ds(s * z_rows, z_rows)],
                    out_hbm.at[c, pl.ds(s * z_rows, z_rows)])

  return sc_agg


def _mlp1_body(x_ref, agg_ref, w1_ref, b1_ref, w2_ref, b2_ref, out_ref):
  hd = agg_ref.shape[2]
  xa = x_ref[...] + jnp.concatenate([agg_ref[0], agg_ref[1]], axis=1)
  h = jnp.dot(xa, w1_ref[...], preferred_element_type=jnp.float32,
              precision=lax.Precision.HIGHEST) + b1_ref[...]
  h = jnp.maximum(h, 0.0)
  h = jnp.dot(h, w2_ref[...], preferred_element_type=jnp.float32,
              precision=lax.Precision.HIGHEST) + b2_ref[...]
  h = jnp.maximum(h, 0.0)
  out_ref[0] = h[:, :hd]
  out_ref[1] = h[:, hd:]


def _mlp2_body(h_ref, agg_ref, w1_ref, b1_ref, w2_ref, b2_ref, bat_ref,
               fcw_ref, fcb_ref, out_ref, acc):
  i = pl.program_id(0)
  g = acc.shape[0]
  r = bat_ref.shape[2]
  xa = (jnp.concatenate([h_ref[0], h_ref[1]], axis=1)
        + jnp.concatenate([agg_ref[0], agg_ref[1]], axis=1))
  h = jnp.dot(xa, w1_ref[...], preferred_element_type=jnp.float32,
              precision=lax.Precision.HIGHEST) + b1_ref[...]
  h = jnp.maximum(h, 0.0)
  h = jnp.dot(h, w2_ref[...], preferred_element_type=jnp.float32,
              precision=lax.Precision.HIGHEST) + b2_ref[...]
  h = jnp.maximum(h, 0.0)
  seg = bat_ref[0, 0, :]
  mask = (seg[:, None] == lax.broadcasted_iota(jnp.int32, (r, g), 1)
          ).astype(jnp.float32)
  part = lax.dot_general(mask, h, (((0,), (0,)), ((), ())),
                         preferred_element_type=jnp.float32,
                         precision=lax.Precision.HIGHEST)

  @pl.when(i == 0)
  def _():
    acc[...] = part

  @pl.when(i > 0)
  def _():
    acc[...] += part

  @pl.when(i == pl.num_programs(0) - 1)
  def _():
    out_ref[...] = jnp.dot(acc[...], fcw_ref[...],
                           preferred_element_type=jnp.float32,
                           precision=lax.Precision.HIGHEST) + fcb_ref[...]


def kernel(x, edge_index, batch, w11, b11, w21, b21, w12, b12, w22, b22,
           fcw, fcb):
  n, d = x.shape
  hdim = w11.shape[1]
  out_dim = fcw.shape[1]
  g = 64
  hd = d // 2
  e = edge_index.shape[1]

  n_chunks = 2 * (-(-e // (NS * K * 2)))
  e_pad = NS * n_chunks * K
  acc_rows = -(-(n + 1) // (NS * 8)) * (NS * 8)

  src = edge_index[0].astype(jnp.int32)
  dst = edge_index[1].astype(jnp.int32)
  pad = e_pad - e
  srcp = jnp.concatenate([src, jnp.zeros((pad,), jnp.int32)])
  dstp = jnp.concatenate([dst, jnp.full((pad,), n, jnp.int32)])
  src4 = jnp.stack([srcp, srcp + n]).reshape(NC, NS, n_chunks, K)
  src4 = jnp.concatenate(
      [src4, jnp.zeros((NC, NS, 1, K), jnp.int32)], axis=2)
  dst4 = dstp.reshape(NS, n_chunks, 1, K)
  dst4 = jnp.concatenate(
      [dst4, jnp.full((NS, 2, 1, K), n, jnp.int32)], axis=1)
  zeros_blk = jnp.zeros((acc_rows // NS, hd), jnp.float32)

  sc_agg = _make_sc_agg(n, hd, n_chunks, acc_rows)

  xs = jnp.concatenate([x[:, :hd], x[:, hd:]], axis=0)

  agg1 = sc_agg(xs, src4, dst4, zeros_blk)

  r = 2000
  n_blocks = n // r
  b11r = b11.reshape(1, -1)
  b21r = b21.reshape(1, -1)
  b12r = b12.reshape(1, -1)
  b22r = b22.reshape(1, -1)
  fcbr = fcb.reshape(1, -1)

  half_spec = pl.BlockSpec((NC, r, hd), lambda i: (0, i, 0))
  wspec = pl.BlockSpec((d, hdim), lambda i: (0, 0))
  bspec = pl.BlockSpec((1, hdim), lambda i: (0, 0))

  h1 = pl.pallas_call(
      _mlp1_body,
      grid=(n_blocks,),
      in_specs=[
          pl.BlockSpec((r, d), lambda i: (i, 0)),
          half_spec,
          wspec, bspec, wspec, bspec,
      ],
      out_specs=half_spec,
      out_shape=jax.ShapeDtypeStruct((NC, n, hd), jnp.float32),
  )(x, agg1, w11, b11r, w21, b21r)

  agg2 = sc_agg(h1.reshape(NC * n, hd), src4, dst4, zeros_blk)

  bat3 = batch.astype(jnp.int32).reshape(n_blocks, 1, r)

  out = pl.pallas_call(
      _mlp2_body,
      grid=(n_blocks,),
      in_specs=[
          half_spec,
          half_spec,
          wspec, bspec, wspec, bspec,
          pl.BlockSpec((1, 1, r), lambda i: (i, 0, 0)),
          pl.BlockSpec((hdim, out_dim), lambda i: (0, 0)),
          pl.BlockSpec((1, out_dim), lambda i: (0, 0)),
      ],
      out_specs=pl.BlockSpec((g, out_dim), lambda i: (0, 0)),
      out_shape=jax.ShapeDtypeStruct((g, out_dim), jnp.float32),
      scratch_shapes=[pltpu.VMEM((g, hdim), jnp.float32)],
  )(h1, agg2, w12, b12r, w22, b22r, bat3, fcw, fcbr)

  return out


# speedup vs baseline: 1.3140x; 1.3140x over previous
import functools

import jax
import jax.numpy as jnp
from jax import lax
from jax.experimental import pallas as pl
from jax.experimental.pallas import tpu as pltpu
from jax.experimental.pallas import tpu_sc as plsc

NC = 2
NS = 16
K = 128


def _make_sc_agg(n, hd, n_chunks, acc_rows):
  z_rows = acc_rows // NS

  mesh = plsc.VectorSubcoreMesh(core_axis_name="c", subcore_axis_name="s")

  @functools.partial(
      pl.kernel,
      out_type=jax.ShapeDtypeStruct((NC, acc_rows, hd), jnp.float32),
      mesh=mesh,
      scratch_types=[
          pltpu.VMEM((n_chunks + 1, K), jnp.int32),
          pltpu.VMEM((1, K), jnp.int32),
          pltpu.VMEM((1, K), jnp.int32),
          pltpu.VMEM((K, hd), jnp.float32),
          pltpu.VMEM((K, hd), jnp.float32),
          pltpu.SemaphoreType.DMA,
          pltpu.SemaphoreType.DMA,
          pltpu.SemaphoreType.DMA,
          pltpu.SemaphoreType.DMA,
          pltpu.VMEM_SHARED((acc_rows, hd), jnp.float32),
      ],
  )
  def sc_agg(xs_hbm, src_hbm, dst_hbm, zeros_hbm, out_hbm,
             srcv, d0, d1, g0, g1, semd0, semd1, semg0, semg1, acc):
    c = lax.axis_index("c")
    s = lax.axis_index("s")
    pltpu.sync_copy(zeros_hbm, acc.at[pl.ds(s * z_rows, z_rows)])
    pltpu.sync_copy(src_hbm.at[c, s], srcv)
    plsc.subcore_barrier()

    pltpu.async_copy(dst_hbm.at[s, 0], d0, semd0)
    pltpu.async_copy(dst_hbm.at[s, 1], d1, semd1)

    def chunk_pair(p, carry):
      j0 = 2 * p
      cg0 = pltpu.async_copy(xs_hbm.at[srcv.at[j0]], g0, semg0)
      cg1 = pltpu.async_copy(xs_hbm.at[srcv.at[j0 + 1]], g1, semg1)
      cg0.wait()
      pltpu.make_async_copy(dst_hbm.at[s, j0], d0, semd0).wait()
      pltpu.sync_copy(g0, acc.at[d0.at[0]], add=True)
      pltpu.async_copy(dst_hbm.at[s, j0 + 2], d0, semd0)
      cg1.wait()
      pltpu.make_async_copy(dst_hbm.at[s, j0 + 1], d1, semd1).wait()
      pltpu.sync_copy(g1, acc.at[d1.at[0]], add=True)
      pltpu.async_copy(dst_hbm.at[s, j0 + 3], d1, semd1)
      return carry

    lax.fori_loop(0, n_chunks // 2, chunk_pair, 0, unroll=False)
    pltpu.make_async_copy(dst_hbm.at[s, 0], d0, semd0).wait()
    pltpu.make_async_copy(dst_hbm.at[s, 1], d1, semd1).wait()
    plsc.subcore_barrier()
    pltpu.sync_copy(acc.at[pl.ds(s * z_rows, z_rows)],
                    out_hbm.at[c, pl.ds(s * z_rows, z_rows)])

  return sc_agg


def _mlp1_body(x_ref, agg_ref, w1_ref, b1_ref, w2_ref, b2_ref, out_ref):
  hd = agg_ref.shape[2]
  xa = x_ref[...] + jnp.concatenate([agg_ref[0], agg_ref[1]], axis=1)
  h = jnp.dot(xa, w1_ref[...], preferred_element_type=jnp.float32,
              precision=lax.Precision.HIGHEST) + b1_ref[...]
  h = jnp.maximum(h, 0.0)
  h = jnp.dot(h, w2_ref[...], preferred_element_type=jnp.float32,
              precision=lax.Precision.HIGHEST) + b2_ref[...]
  h = jnp.maximum(h, 0.0)
  out_ref[0] = h[:, :hd]
  out_ref[1] = h[:, hd:]


def _mlp2_body(h_ref, agg_ref, w1_ref, b1_ref, w2_ref, b2_ref, bat_ref,
               fcw_ref, fcb_ref, out_ref, acc):
  i = pl.program_id(0)
  g = acc.shape[0]
  r = bat_ref.shape[2]
  xa = (jnp.concatenate([h_ref[0], h_ref[1]], axis=1)
        + jnp.concatenate([agg_ref[0], agg_ref[1]], axis=1))
  h = jnp.dot(xa, w1_ref[...], preferred_element_type=jnp.float32,
              precision=lax.Precision.HIGHEST) + b1_ref[...]
  h = jnp.maximum(h, 0.0)
  h = jnp.dot(h, w2_ref[...], preferred_element_type=jnp.float32,
              precision=lax.Precision.HIGHEST) + b2_ref[...]
  h = jnp.maximum(h, 0.0)
  seg = bat_ref[0, 0, :]
  mask = (seg[:, None] == lax.broadcasted_iota(jnp.int32, (r, g), 1)
          ).astype(jnp.float32)
  part = lax.dot_general(mask, h, (((0,), (0,)), ((), ())),
                         preferred_element_type=jnp.float32,
                         precision=lax.Precision.HIGHEST)

  @pl.when(i == 0)
  def _():
    acc[...] = part

  @pl.when(i > 0)
  def _():
    acc[...] += part

  @pl.when(i == pl.num_programs(0) - 1)
  def _():
    out_ref[...] = jnp.dot(acc[...], fcw_ref[...],
                           preferred_element_type=jnp.float32,
                           precision=lax.Precision.HIGHEST) + fcb_ref[...]


def kernel(x, edge_index, batch, w11, b11, w21, b21, w12, b12, w22, b22,
           fcw, fcb):
  n, d = x.shape
  hdim = w11.shape[1]
  out_dim = fcw.shape[1]
  g = 64
  hd = d // 2
  e = edge_index.shape[1]

  n_chunks = 2 * (-(-e // (NS * K * 2)))
  e_pad = NS * n_chunks * K
  acc_rows = -(-(n + 1) // (NS * 8)) * (NS * 8)

  src = edge_index[0].astype(jnp.int32)
  dst = edge_index[1].astype(jnp.int32)
  pad = e_pad - e
  srcp = jnp.concatenate([src, jnp.zeros((pad,), jnp.int32)])
  dstp = jnp.concatenate([dst, jnp.full((pad,), n, jnp.int32)])
  src4 = jnp.stack([srcp, srcp + n]).reshape(NC, NS, n_chunks, K)
  src4 = jnp.concatenate(
      [src4, jnp.zeros((NC, NS, 1, K), jnp.int32)], axis=2)
  dst4 = dstp.reshape(NS, n_chunks, 1, K)
  dst4 = jnp.concatenate(
      [dst4, jnp.full((NS, 2, 1, K), n, jnp.int32)], axis=1)
  zeros_blk = jnp.zeros((acc_rows // NS, hd), jnp.float32)

  sc_agg = _make_sc_agg(n, hd, n_chunks, acc_rows)

  xs = jnp.concatenate([x[:, :hd], x[:, hd:]], axis=0)

  agg1 = sc_agg(xs, src4, dst4, zeros_blk)

  r = 2000
  n_blocks = n // r
  b11r = b11.reshape(1, -1)
  b21r = b21.reshape(1, -1)
  b12r = b12.reshape(1, -1)
  b22r = b22.reshape(1, -1)
  fcbr = fcb.reshape(1, -1)

  half_spec = pl.BlockSpec((NC, r, hd), lambda i: (0, i, 0))
  wspec = pl.BlockSpec((d, hdim), lambda i: (0, 0))
  bspec = pl.BlockSpec((1, hdim), lambda i: (0, 0))

  h1 = pl.pallas_call(
      _mlp1_body,
      grid=(n_blocks,),
      in_specs=[
          pl.BlockSpec((r, d), lambda i: (i, 0)),
          half_spec,
          wspec, bspec, wspec, bspec,
      ],
      out_specs=half_spec,
      out_shape=jax.ShapeDtypeStruct((NC, n, hd), jnp.float32),
  )(x, agg1, w11, b11r, w21, b21r)

  agg2 = sc_agg(h1.reshape(NC * n, hd), src4, dst4, zeros_blk)

  bat3 = batch.astype(jnp.int32).reshape(n_blocks, 1, r)

  out = pl.pallas_call(
      _mlp2_body,
      grid=(n_blocks,),
      in_specs=[
          half_spec,
          half_spec,
          wspec, bspec, wspec, bspec,
          pl.BlockSpec((1, 1, r), lambda i: (i, 0, 0)),
          pl.BlockSpec((hdim, out_dim), lambda i: (0, 0)),
          pl.BlockSpec((1, out_dim), lambda i: (0, 0)),
      ],
      out_specs=pl.BlockSpec((g, out_dim), lambda i: (0, 0)),
      out_shape=jax.ShapeDtypeStruct((g, out_dim), jnp.float32),
      scratch_shapes=[pltpu.VMEM((g, hdim), jnp.float32)],
  )(h1, agg2, w12, b12r, w22, b22r, bat3, fcw, fcbr)

  return out


# speedup vs baseline: 1.7827x; 1.3567x over previous
import functools

import jax
import jax.numpy as jnp
from jax import lax
from jax.experimental import pallas as pl
from jax.experimental.pallas import tpu as pltpu
from jax.experimental.pallas import tpu_sc as plsc

NC = 2
NS = 16
K = 128


def _make_sc_agg(n, hd, n_chunks, acc_rows):
  z_rows = acc_rows // NS

  mesh = plsc.VectorSubcoreMesh(core_axis_name="c", subcore_axis_name="s")

  @functools.partial(
      pl.kernel,
      out_type=jax.ShapeDtypeStruct((NC, acc_rows, hd), jnp.float32),
      mesh=mesh,
      scratch_types=[
          pltpu.VMEM((n_chunks, K), jnp.int32),
          pltpu.VMEM((n_chunks, K), jnp.int32),
          pltpu.VMEM((K, hd), jnp.float32),
          pltpu.SemaphoreType.DMA,
          pltpu.VMEM_SHARED((acc_rows, hd), jnp.float32),
      ],
  )
  def sc_agg(xs_hbm, src_hbm, dst_hbm, zeros_hbm, out_hbm,
             srcv, dstv, gbuf, sem, acc):
    c = lax.axis_index("c")
    s = lax.axis_index("s")
    pltpu.sync_copy(zeros_hbm, acc.at[pl.ds(s * z_rows, z_rows)])
    pltpu.sync_copy(src_hbm.at[c, s], srcv)
    pltpu.sync_copy(dst_hbm.at[s], dstv)
    plsc.subcore_barrier()

    def chunk(j, carry):
      pltpu.async_copy(xs_hbm.at[srcv.at[j]], gbuf, sem).wait()
      pltpu.sync_copy(gbuf, acc.at[dstv.at[j]], add=True)
      return carry

    lax.fori_loop(0, n_chunks, chunk, 0, unroll=False)
    plsc.subcore_barrier()
    pltpu.sync_copy(acc.at[pl.ds(s * z_rows, z_rows)],
                    out_hbm.at[c, pl.ds(s * z_rows, z_rows)])

  return sc_agg


def _mlp1_body(x_ref, agg_ref, w1_ref, b1_ref, w2_ref, b2_ref, out_ref):
  hd = agg_ref.shape[2]
  xa = x_ref[...] + jnp.concatenate([agg_ref[0], agg_ref[1]], axis=1)
  h = jnp.dot(xa, w1_ref[...], preferred_element_type=jnp.float32) + b1_ref[...]
  h = jnp.maximum(h, 0.0)
  h = jnp.dot(h, w2_ref[...], preferred_element_type=jnp.float32) + b2_ref[...]
  h = jnp.maximum(h, 0.0)
  out_ref[0] = h[:, :hd]
  out_ref[1] = h[:, hd:]


def _mlp2_body(h_ref, agg_ref, w1_ref, b1_ref, w2_ref, b2_ref, bat_ref,
               fcw_ref, fcb_ref, out_ref, acc):
  i = pl.program_id(0)
  g = acc.shape[0]
  r = bat_ref.shape[2]
  xa = (jnp.concatenate([h_ref[0], h_ref[1]], axis=1)
        + jnp.concatenate([agg_ref[0], agg_ref[1]], axis=1))
  h = jnp.dot(xa, w1_ref[...], preferred_element_type=jnp.float32) + b1_ref[...]
  h = jnp.maximum(h, 0.0)
  h = jnp.dot(h, w2_ref[...], preferred_element_type=jnp.float32) + b2_ref[...]
  h = jnp.maximum(h, 0.0)
  seg = bat_ref[0, 0, :]
  mask = (seg[:, None] == lax.broadcasted_iota(jnp.int32, (r, g), 1)
          ).astype(jnp.float32)
  part = lax.dot_general(mask, h, (((0,), (0,)), ((), ())),
                         preferred_element_type=jnp.float32)

  @pl.when(i == 0)
  def _():
    acc[...] = part

  @pl.when(i > 0)
  def _():
    acc[...] += part

  @pl.when(i == pl.num_programs(0) - 1)
  def _():
    out_ref[...] = jnp.dot(acc[...], fcw_ref[...],
                           preferred_element_type=jnp.float32) + fcb_ref[...]


def kernel(x, edge_index, batch, w11, b11, w21, b21, w12, b12, w22, b22,
           fcw, fcb):
  n, d = x.shape
  hdim = w11.shape[1]
  out_dim = fcw.shape[1]
  g = 64
  hd = d // 2
  e = edge_index.shape[1]

  n_chunks = -(-e // (NS * K))
  e_pad = NS * n_chunks * K
  acc_rows = -(-(n + 1) // (NS * 8)) * (NS * 8)

  src = edge_index[0].astype(jnp.int32)
  dst = edge_index[1].astype(jnp.int32)
  pad = e_pad - e
  srcp = jnp.concatenate([src, jnp.zeros((pad,), jnp.int32)])
  dstp = jnp.concatenate([dst, jnp.full((pad,), n, jnp.int32)])
  src4 = jnp.stack([srcp, srcp + n]).reshape(NC, NS, n_chunks, K)
  dst3 = dstp.reshape(NS, n_chunks, K)
  zeros_blk = jnp.zeros((acc_rows // NS, hd), jnp.float32)

  sc_agg = _make_sc_agg(n, hd, n_chunks, acc_rows)

  xs = jnp.concatenate([x[:, :hd], x[:, hd:]], axis=0)

  agg1 = sc_agg(xs, src4, dst3, zeros_blk)

  r = 2000
  n_blocks = n // r
  b11r = b11.reshape(1, -1)
  b21r = b21.reshape(1, -1)
  b12r = b12.reshape(1, -1)
  b22r = b22.reshape(1, -1)
  fcbr = fcb.reshape(1, -1)

  half_spec = pl.BlockSpec((NC, r, hd), lambda i: (0, i, 0))
  wspec = pl.BlockSpec((d, hdim), lambda i: (0, 0))
  bspec = pl.BlockSpec((1, hdim), lambda i: (0, 0))

  h1 = pl.pallas_call(
      _mlp1_body,
      grid=(n_blocks,),
      in_specs=[
          pl.BlockSpec((r, d), lambda i: (i, 0)),
          half_spec,
          wspec, bspec, wspec, bspec,
      ],
      out_specs=half_spec,
      out_shape=jax.ShapeDtypeStruct((NC, n, hd), jnp.float32),
  )(x, agg1, w11, b11r, w21, b21r)

  agg2 = sc_agg(h1.reshape(NC * n, hd), src4, dst3, zeros_blk)

  bat3 = batch.astype(jnp.int32).reshape(n_blocks, 1, r)

  out = pl.pallas_call(
      _mlp2_body,
      grid=(n_blocks,),
      in_specs=[
          half_spec,
          half_spec,
          wspec, bspec, wspec, bspec,
          pl.BlockSpec((1, 1, r), lambda i: (i, 0, 0)),
          pl.BlockSpec((hdim, out_dim), lambda i: (0, 0)),
          pl.BlockSpec((1, out_dim), lambda i: (0, 0)),
      ],
      out_specs=pl.BlockSpec((g, out_dim), lambda i: (0, 0)),
      out_shape=jax.ShapeDtypeStruct((g, out_dim), jnp.float32),
      scratch_shapes=[pltpu.VMEM((g, hdim), jnp.float32)],
  )(h1, agg2, w12, b12r, w22, b22r, bat3, fcw, fcbr)

  return out
